# Initial kernel scaffold; baseline (speedup 1.0000x reference)
import jax
import jax.numpy as jnp
from jax.experimental import pallas as pl


def kernel(x, gamma, beta, w1, b1, w2, b2):
    raise NotImplementedError("write your pallas kernel here")


# speedup vs baseline: 1.0250x; 1.0250x over previous
import functools

import jax
import jax.numpy as jnp
from jax.experimental import pallas as pl
from jax.experimental.pallas import tpu as pltpu


def _ffn_kernel(x_ref, g_ref, bt_ref, w1_ref, b1_ref, w2_ref, b2_ref, o_ref,
                *, eps):
    x = x_ref[...].astype(jnp.float32)
    mean = jnp.mean(x, axis=-1, keepdims=True)
    xc = x - mean
    var = jnp.mean(xc * xc, axis=-1, keepdims=True)
    xn = xc * jax.lax.rsqrt(var + eps)
    xn = xn * g_ref[...] + bt_ref[...]

    h = jnp.dot(xn.astype(jnp.bfloat16), w1_ref[...],
                preferred_element_type=jnp.float32)
    h = jax.nn.gelu(h + b1_ref[...], approximate=True)

    y = jnp.dot(h.astype(jnp.bfloat16), w2_ref[...],
                preferred_element_type=jnp.float32)
    o_ref[...] = (y + b2_ref[...]).astype(o_ref.dtype)


def kernel(x, gamma, beta, w1, b1, w2, b2, *, eps=1e-5, row_tile=256):
    B, S, E = x.shape
    M = w1.shape[1]
    R = B * S

    x2 = x.reshape(R, E)
    R_pad = ((max(R, 1) + row_tile - 1) // row_tile) * row_tile
    if R_pad != R:
        x2 = jnp.pad(x2, ((0, R_pad - R), (0, 0)))
    n_tiles = R_pad // row_tile

    w1b = w1.astype(jnp.bfloat16)
    w2b = w2.astype(jnp.bfloat16)
    g2 = gamma.reshape(1, E).astype(jnp.float32)
    bt2 = beta.reshape(1, E).astype(jnp.float32)
    b1_2 = b1.reshape(1, M).astype(jnp.float32)
    b2_2 = b2.reshape(1, E).astype(jnp.float32)

    out = pl.pallas_call(
        functools.partial(_ffn_kernel, eps=eps),
        out_shape=jax.ShapeDtypeStruct((R_pad, E), x.dtype),
        grid=(n_tiles,),
        in_specs=[
            pl.BlockSpec((row_tile, E), lambda i: (i, 0)),
            pl.BlockSpec((1, E), lambda i: (0, 0)),
            pl.BlockSpec((1, E), lambda i: (0, 0)),
            pl.BlockSpec((E, M), lambda i: (0, 0)),
            pl.BlockSpec((1, M), lambda i: (0, 0)),
            pl.BlockSpec((M, E), lambda i: (0, 0)),
            pl.BlockSpec((1, E), lambda i: (0, 0)),
        ],
        out_specs=pl.BlockSpec((row_tile, E), lambda i: (i, 0)),
        compiler_params=pltpu.CompilerParams(
            dimension_semantics=("parallel",),
            vmem_limit_bytes=56 * 1024 * 1024,
        ),
        cost_estimate=pl.CostEstimate(
            flops=int(4 * R * E * M),
            transcendentals=int(R * M),
            bytes_accessed=int(R * E * 4 + R * E * 4 + 2 * E * M * 2),
        ),
    )(x2, g2, bt2, w1b, b1_2, w2b, b2_2)

    return out[:R].reshape(B, S, E)


# speedup vs baseline: 1.0809x; 1.0545x over previous
import functools

import jax
import jax.numpy as jnp
from jax.experimental import pallas as pl
from jax.experimental.pallas import tpu as pltpu


def _ffn_kernel(x_ref, g_ref, bt_ref, w1_ref, b1_ref, w2_ref, b2_ref, o_ref,
                *, eps):
    x = x_ref[...].astype(jnp.float32)
    mean = jnp.mean(x, axis=-1, keepdims=True)
    xc = x - mean
    var = jnp.mean(xc * xc, axis=-1, keepdims=True)
    xn = xc * jax.lax.rsqrt(var + eps)
    xn = xn * g_ref[...] + bt_ref[...]

    h = jnp.dot(xn.astype(jnp.bfloat16), w1_ref[...],
                preferred_element_type=jnp.float32)
    h = jax.nn.gelu(h + b1_ref[...], approximate=True)

    y = jnp.dot(h.astype(jnp.bfloat16), w2_ref[...],
                preferred_element_type=jnp.float32)
    o_ref[...] = (y + b2_ref[...]).astype(o_ref.dtype)


def kernel(x, gamma, beta, w1, b1, w2, b2, *, eps=1e-5, row_tile=512):
    B, S, E = x.shape
    M = w1.shape[1]
    R = B * S

    x2 = x.reshape(R, E)
    tile2 = 2 * row_tile
    R_pad = ((max(R, 1) + tile2 - 1) // tile2) * tile2
    if R_pad != R:
        x2 = jnp.pad(x2, ((0, R_pad - R), (0, 0)))
    T = R_pad // row_tile // 2

    w1b = w1.astype(jnp.bfloat16)
    w2b = w2.astype(jnp.bfloat16)
    g2 = gamma.reshape(1, E).astype(jnp.float32)
    bt2 = beta.reshape(1, E).astype(jnp.float32)
    b1_2 = b1.reshape(1, M).astype(jnp.float32)
    b2_2 = b2.reshape(1, E).astype(jnp.float32)

    out = pl.pallas_call(
        functools.partial(_ffn_kernel, eps=eps),
        out_shape=jax.ShapeDtypeStruct((R_pad, E), x.dtype),
        grid=(2, T),
        in_specs=[
            pl.BlockSpec((row_tile, E), lambda i, k: (i * T + k, 0)),
            pl.BlockSpec((1, E), lambda i, k: (0, 0)),
            pl.BlockSpec((1, E), lambda i, k: (0, 0)),
            pl.BlockSpec((E, M), lambda i, k: (0, 0),
                         pipeline_mode=pl.Buffered(1)),
            pl.BlockSpec((1, M), lambda i, k: (0, 0)),
            pl.BlockSpec((M, E), lambda i, k: (0, 0),
                         pipeline_mode=pl.Buffered(1)),
            pl.BlockSpec((1, E), lambda i, k: (0, 0)),
        ],
        out_specs=pl.BlockSpec((row_tile, E), lambda i, k: (i * T + k, 0)),
        compiler_params=pltpu.CompilerParams(
            dimension_semantics=("arbitrary", "arbitrary"),
            vmem_limit_bytes=56 * 1024 * 1024,
        ),
        cost_estimate=pl.CostEstimate(
            flops=int(4 * R * E * M),
            transcendentals=int(R * M),
            bytes_accessed=int(R * E * 4 + R * E * 4 + 2 * E * M * 2),
        ),
    )(x2, g2, bt2, w1b, b1_2, w2b, b2_2)

    return out[:R].reshape(B, S, E)


# speedup vs baseline: 1.1706x; 1.0830x over previous
import functools

import jax
import jax.numpy as jnp
from jax.experimental import pallas as pl
from jax.experimental.pallas import tpu as pltpu


def _ffn_kernel(x_ref, g_ref, bt_ref, w1_ref, b1_ref, w2_ref, b2_ref, o_ref,
                w1b_ref, w2b_ref, st1_ref, st2_ref, sem1_ref, sem2_ref,
                *, eps, c1, c2):
    E = w1_ref.shape[0]
    M = w2_ref.shape[0]
    n1 = E // c1
    n2 = M // c2

    @pl.when(pl.program_id(0) == 0)
    def _load_weights():
        def start1(c, buf):
            pltpu.make_async_copy(w1_ref.at[pl.ds(c * c1, c1), :],
                                  st1_ref.at[buf], sem1_ref.at[buf]).start()

        def start2(c, buf):
            pltpu.make_async_copy(w2_ref.at[pl.ds(c * c2, c2), :],
                                  st2_ref.at[buf], sem2_ref.at[buf]).start()

        start1(0, 0)
        if n1 > 1:
            start1(1, 1)
        n2_started = 0
        for c in range(n1):
            buf = c % 2
            pltpu.make_async_copy(st1_ref.at[buf], st1_ref.at[buf],
                                  sem1_ref.at[buf]).wait()
            if c + 2 < n1:
                start1(c + 2, buf)
            elif n2_started < min(2, n2):
                start2(n2_started, n2_started)
                n2_started += 1
            w1b_ref[pl.ds(c * c1, c1), :] = st1_ref[buf].astype(jnp.bfloat16)
        while n2_started < min(2, n2):
            start2(n2_started, n2_started)
            n2_started += 1
        for c in range(n2):
            buf = c % 2
            pltpu.make_async_copy(st2_ref.at[buf], st2_ref.at[buf],
                                  sem2_ref.at[buf]).wait()
            if c + 2 < n2:
                start2(c + 2, buf)
            w2b_ref[pl.ds(c * c2, c2), :] = st2_ref[buf].astype(jnp.bfloat16)

    x = x_ref[...].astype(jnp.float32)
    mean = jnp.mean(x, axis=-1, keepdims=True)
    xc = x - mean
    var = jnp.mean(xc * xc, axis=-1, keepdims=True)
    xn = xc * jax.lax.rsqrt(var + eps)
    xn = xn * g_ref[...] + bt_ref[...]

    h = jnp.dot(xn.astype(jnp.bfloat16), w1b_ref[...],
                preferred_element_type=jnp.float32)
    h = jax.nn.gelu(h + b1_ref[...], approximate=True)

    y = jnp.dot(h.astype(jnp.bfloat16), w2b_ref[...],
                preferred_element_type=jnp.float32)
    o_ref[...] = (y + b2_ref[...]).astype(o_ref.dtype)


def kernel(x, gamma, beta, w1, b1, w2, b2, *, eps=1e-5, row_tile=512,
           c1=128, c2=512, interpret=False):
    B, S, E = x.shape
    M = w1.shape[1]
    R = B * S

    x2 = x.reshape(R, E)
    R_pad = ((max(R, 1) + row_tile - 1) // row_tile) * row_tile
    if R_pad != R:
        x2 = jnp.pad(x2, ((0, R_pad - R), (0, 0)))
    T = R_pad // row_tile

    g2 = gamma.reshape(1, E).astype(jnp.float32)
    bt2 = beta.reshape(1, E).astype(jnp.float32)
    b1_2 = b1.reshape(1, M).astype(jnp.float32)
    b2_2 = b2.reshape(1, E).astype(jnp.float32)

    out = pl.pallas_call(
        functools.partial(_ffn_kernel, eps=eps, c1=c1, c2=c2),
        out_shape=jax.ShapeDtypeStruct((R_pad, E), x.dtype),
        grid=(T,),
        in_specs=[
            pl.BlockSpec((row_tile, E), lambda i: (i, 0)),
            pl.BlockSpec((1, E), lambda i: (0, 0)),
            pl.BlockSpec((1, E), lambda i: (0, 0)),
            pl.BlockSpec(memory_space=pl.ANY),
            pl.BlockSpec((1, M), lambda i: (0, 0)),
            pl.BlockSpec(memory_space=pl.ANY),
            pl.BlockSpec((1, E), lambda i: (0, 0)),
        ],
        out_specs=pl.BlockSpec((row_tile, E), lambda i: (i, 0)),
        scratch_shapes=[
            pltpu.VMEM((E, M), jnp.bfloat16),
            pltpu.VMEM((M, E), jnp.bfloat16),
            pltpu.VMEM((2, c1, M), jnp.float32),
            pltpu.VMEM((2, c2, E), jnp.float32),
            pltpu.SemaphoreType.DMA((2,)),
            pltpu.SemaphoreType.DMA((2,)),
        ],
        compiler_params=pltpu.CompilerParams(
            dimension_semantics=("arbitrary",),
            vmem_limit_bytes=56 * 1024 * 1024,
        ),
        cost_estimate=pl.CostEstimate(
            flops=int(4 * R * E * M),
            transcendentals=int(R * M),
            bytes_accessed=int(R * E * 4 + R * E * 4 + 2 * E * M * 4),
        ),
        interpret=interpret,
    )(x2, g2, bt2, w1, b1_2, w2, b2_2)

    return out[:R].reshape(B, S, E)
